# Initial kernel scaffold; baseline (speedup 1.0000x reference)
import jax
import jax.numpy as jnp
from jax.experimental import pallas as pl


def kernel(x, edge_index, batch, W1, b1, W2, b2, W3, b3, Wl, bl):
    raise NotImplementedError("write your pallas kernel here")


# speedup vs baseline: 5.8338x; 5.8338x over previous
import functools

import jax
import jax.numpy as jnp
from jax import lax
from jax.experimental import pallas as pl
from jax.experimental.pallas import tpu as pltpu
from jax.experimental.pallas import tpu_sc as plsc

NN = 100000
NP = 107136
SR = NP // 16
EE = 1600000
CH = 128
ECH = 512
NRANGE = 9
RNG = NP // NRANGE
ACCR = RNG + 128
ZSTR = ACCR // 16
DSTR = RNG // 16
RB = 4464
GRID = NP // RB
F32 = jnp.float32

_MESH = plsc.VectorSubcoreMesh(core_axis_name="c", subcore_axis_name="s")


def _iota16():
    return lax.iota(jnp.int32, 16)


def _sc_deg(dst):
    kern = pl.kernel(
        _sc_deg_body,
        out_type=jax.ShapeDtypeStruct((2 * NP,), F32),
        mesh=_MESH,
        scratch_types=[
            pltpu.VMEM((CH,), jnp.int32),
            pltpu.VMEM((CH,), F32),
            pltpu.VMEM((SR,), F32),
            pltpu.VMEM_SHARED((NP,), F32),
        ],
    )
    return kern(dst)


def _sc_deg_body(dst_hbm, out_hbm, didx, ones_b, zb, acc):
    c = lax.axis_index("c")
    s = lax.axis_index("s")
    for j in range(CH // 16):
        ones_b[pl.ds(j * 16, 16)] = jnp.full((16,), 1.0, F32)

    def zrow(i, _):
        zb[pl.ds(i * 16, 16)] = jnp.zeros((16,), F32)
        return _

    lax.fori_loop(0, SR // 16, zrow, None)
    pltpu.sync_copy(zb, acc.at[pl.ds(s * SR, SR)])
    plsc.subcore_barrier()

    per_core = EE // 2
    per_sub = per_core // 16
    nfull = per_sub // CH
    tail = per_sub - nfull * CH
    base = c * per_core + s * per_sub

    def chunk(i, _):
        pltpu.sync_copy(dst_hbm.at[pl.ds(base + i * CH, CH)], didx)
        pltpu.sync_copy(ones_b, acc.at[didx], add=True)
        return _

    lax.fori_loop(0, nfull, chunk, None)
    pltpu.sync_copy(dst_hbm.at[pl.ds(base + nfull * CH, tail)],
                    didx.at[pl.ds(0, tail)])
    for j in range(tail // 16, CH // 16):
        didx[pl.ds(j * 16, 16)] = NN + _iota16()
    pltpu.sync_copy(ones_b, acc.at[didx], add=True)

    plsc.subcore_barrier()
    pltpu.sync_copy(acc.at[pl.ds(s * SR, SR)], zb)
    pltpu.sync_copy(zb, out_hbm.at[pl.ds(c * NP + s * SR, SR)])


BIGDST = jnp.int32(1 << 20)


def _sc_agg(table, src, dst):
    kern = pl.kernel(
        _sc_agg_body,
        out_type=jax.ShapeDtypeStruct((2, NRANGE, RNG, 128), F32),
        mesh=_MESH,
        compiler_params=pltpu.CompilerParams(needs_layout_passes=False),
        scratch_types=[
            pltpu.VMEM((ECH,), jnp.int32),
            pltpu.VMEM((ECH,), jnp.int32),
            pltpu.VMEM((2, 128), jnp.int32),
            pltpu.VMEM((2, 128), jnp.int32),
            pltpu.VMEM((128, 128), F32),
            pltpu.VMEM((128, 128), F32),
            pltpu.VMEM_SHARED((ACCR, 128), F32),
            pltpu.SemaphoreType.DMA,
        ],
    )
    return kern(table, src, dst)


def _sc_agg_body(table_hbm, src_hbm, dst_hbm, out_hbm,
                 sbuf, dbuf, cs, cd, rows, zbuf, acc, sem):
    c = lax.axis_index("c")
    s = lax.axis_index("s")
    i32 = jnp.int32

    def zrow(i, _):
        for j in range(8):
            zbuf[i, pl.ds(j * 16, 16)] = jnp.zeros((16,), F32)
        return _

    lax.fori_loop(0, 128, zrow, None)

    per_core = EE // 2
    per_sub = per_core // 16
    nfull = per_sub // ECH
    tail = per_sub - nfull * ECH
    ebase = c * per_core + s * per_sub

    def group(lo, hi, sb_g, db_g, carry):
        cnt, fired = carry
        d16 = db_g
        m = (d16 >= lo) & (d16 < hi)
        pref = plsc.cumsum(jnp.where(m, 1, 0).astype(i32))
        pos = (pref + (jnp.zeros((16,), i32) + cnt) - 1) & 255
        row = (pos >> 7) & 1
        col = pos & 127
        plsc.store_scatter(cs, [row, col], sb_g, mask=m)
        plsc.store_scatter(cd, [row, col], d16 - lo, mask=m)
        cnt = cnt + jnp.sum(jnp.where(m, 1, 0).astype(i32))

        @pl.when(cnt - fired >= 128)
        def _():
            half = (fired // 128) & 1
            pltpu.async_copy(table_hbm.at[cs.at[half]], rows, sem).wait()
            pltpu.sync_copy(rows, acc.at[cd.at[half]], add=True)

        fired = jnp.where(cnt - fired >= 128, fired + 128, fired)
        return (cnt, fired)

    def range_body(r, _):
        lo = r * RNG
        hi = lo + RNG

        zb = s * ZSTR

        def zc(k, __):
            pltpu.sync_copy(zbuf, acc.at[pl.ds(zb + k * 128, 128)])
            return __

        lax.fori_loop(0, ZSTR // 128, zc, None)
        pltpu.sync_copy(zbuf.at[pl.ds(0, ZSTR % 128)],
                        acc.at[pl.ds(zb + (ZSTR // 128) * 128, ZSTR % 128)])
        plsc.subcore_barrier()

        def chunk(i, carry):
            eb = ebase + i * ECH
            pltpu.sync_copy(src_hbm.at[pl.ds(eb, ECH)], sbuf)
            pltpu.sync_copy(dst_hbm.at[pl.ds(eb, ECH)], dbuf)
            for g in range(ECH // 16):
                carry = group(lo, hi, sbuf[pl.ds(g * 16, 16)],
                              dbuf[pl.ds(g * 16, 16)], carry)
            return carry

        carry = lax.fori_loop(0, nfull, chunk, (i32(0), i32(0)))

        tb = ebase + nfull * ECH
        pltpu.sync_copy(src_hbm.at[pl.ds(tb, tail)], sbuf.at[pl.ds(0, tail)])
        pltpu.sync_copy(dst_hbm.at[pl.ds(tb, tail)], dbuf.at[pl.ds(0, tail)])
        for g in range(tail // 16, ECH // 16):
            sbuf[pl.ds(g * 16, 16)] = jnp.zeros((16,), jnp.int32)
            dbuf[pl.ds(g * 16, 16)] = jnp.zeros((16,), jnp.int32) + BIGDST

        def tailchunk(i, carry):
            for g in range(ECH // 16):
                carry = group(lo, hi, sbuf[pl.ds(g * 16, 16)],
                              dbuf[pl.ds(g * 16, 16)], carry)
            return carry

        cnt, fired = lax.fori_loop(0, 1, tailchunk, carry)

        @pl.when(cnt - fired > 0)
        def _():
            limit = jnp.zeros((16,), i32) + (fired + 128)
            for j in range(8):
                posj = (jnp.zeros((16,), i32) + cnt) + j * 16 + _iota16()
                mj = posj < limit
                pw = posj & 255
                plsc.store_scatter(cs, [(pw >> 7) & 1, pw & 127],
                                   _iota16() + (j % 2) * 16, mask=mj)
                plsc.store_scatter(cd, [(pw >> 7) & 1, pw & 127],
                                   RNG + _iota16() + (j % 2) * 16, mask=mj)
            half = (fired // 128) & 1
            pltpu.async_copy(table_hbm.at[cs.at[half]], rows, sem).wait()
            pltpu.sync_copy(rows, acc.at[cd.at[half]], add=True)

        plsc.subcore_barrier()

        db = s * DSTR

        def dc(k, __):
            pltpu.sync_copy(acc.at[pl.ds(db + k * 128, 128)], rows)
            pltpu.sync_copy(rows, out_hbm.at[c, r, pl.ds(db + k * 128, 128)])
            return __

        lax.fori_loop(0, DSTR // 128, dc, None)
        pltpu.sync_copy(acc.at[pl.ds(db + (DSTR // 128) * 128, DSTR % 128)],
                        rows.at[pl.ds(0, DSTR % 128)])
        pltpu.sync_copy(rows.at[pl.ds(0, DSTR % 128)],
                        out_hbm.at[c, r, pl.ds(db + (DSTR // 128) * 128,
                                               DSTR % 128)])
        plsc.subcore_barrier()
        return _

    lax.fori_loop(0, NRANGE, range_body, None)


def _tc_prep(degp, xp):
    def body(d_ref, x_ref, dis_ref, xs_ref):
        deg = d_ref[0] + d_ref[1] + 1.0
        dis = lax.rsqrt(deg)
        dis_ref[...] = dis
        xs_ref[...] = dis * x_ref[...]

    return pl.pallas_call(
        body,
        grid=(GRID,),
        in_specs=[
            pl.BlockSpec((2, RB, 1), lambda i: (0, i, 0)),
            pl.BlockSpec((RB, 128), lambda i: (i, 0)),
        ],
        out_specs=[
            pl.BlockSpec((RB, 1), lambda i: (i, 0)),
            pl.BlockSpec((RB, 128), lambda i: (i, 0)),
        ],
        out_shape=[
            jax.ShapeDtypeStruct((NP, 1), F32),
            jax.ShapeDtypeStruct((NP, 128), F32),
        ],
    )(degp, xp)


def _tc_layer1(y0, xs, dis, w1p, b1r):
    def body(y_ref, xs_ref, dis_ref, w_ref, b_ref, out_ref):
        dis = dis_ref[...]
        ysum = y_ref[0] + y_ref[1]
        agg = dis * (ysum[:, :16] + xs_ref[...][:, :16])
        h = jnp.maximum(
            jnp.dot(agg, w_ref[...], preferred_element_type=F32) + b_ref[...],
            0.0)
        out_ref[...] = jnp.concatenate(
            [dis * h, jnp.zeros((RB, 64), F32)], axis=1)

    return pl.pallas_call(
        body,
        grid=(GRID,),
        in_specs=[
            pl.BlockSpec((2, RB, 128), lambda i: (0, i, 0)),
            pl.BlockSpec((RB, 128), lambda i: (i, 0)),
            pl.BlockSpec((RB, 1), lambda i: (i, 0)),
            pl.BlockSpec((16, 64), lambda i: (0, 0)),
            pl.BlockSpec((1, 64), lambda i: (0, 0)),
        ],
        out_specs=pl.BlockSpec((RB, 128), lambda i: (i, 0)),
        out_shape=jax.ShapeDtypeStruct((NP, 128), F32),
    )(y0, xs, dis, w1p, b1r)


def _tc_layer2(y1, hs1, dis, w2, b2r):
    def body(y_ref, h_ref, dis_ref, w_ref, b_ref, out_ref):
        dis = dis_ref[...]
        ysum = y_ref[0] + y_ref[1]
        agg = dis * (ysum[:, :64] + h_ref[...][:, :64])
        h = jnp.maximum(
            jnp.dot(agg, w_ref[...], preferred_element_type=F32) + b_ref[...],
            0.0)
        out_ref[...] = jnp.concatenate(
            [dis * h, jnp.zeros((RB, 64), F32)], axis=1)

    return pl.pallas_call(
        body,
        grid=(GRID,),
        in_specs=[
            pl.BlockSpec((2, RB, 128), lambda i: (0, i, 0)),
            pl.BlockSpec((RB, 128), lambda i: (i, 0)),
            pl.BlockSpec((RB, 1), lambda i: (i, 0)),
            pl.BlockSpec((64, 64), lambda i: (0, 0)),
            pl.BlockSpec((1, 64), lambda i: (0, 0)),
        ],
        out_specs=pl.BlockSpec((RB, 128), lambda i: (i, 0)),
        out_shape=jax.ShapeDtypeStruct((NP, 128), F32),
    )(y1, hs1, dis, w2, b2r)


def _tc_pool(y3, hsc2, dis, batchp, w3, b3r, wl, blr):
    def body(y_ref, h_ref, dis_ref, b_ref, w3_ref, b3_ref, wl_ref, bl_ref,
             acc_ref, out_ref):
        i = pl.program_id(0)
        dis = dis_ref[...]
        ysum = y_ref[0] + y_ref[1]
        agg = dis * (ysum[:, :64] + h_ref[...][:, :64])
        vals = jnp.concatenate(
            [agg, jnp.ones((RB, 1), F32), jnp.zeros((RB, 63), F32)], axis=1)
        seg = jnp.broadcast_to(b_ref[...], (RB, 64))
        oh = (seg == lax.broadcasted_iota(jnp.int32, (RB, 64), 1)).astype(F32)
        contrib = lax.dot_general(oh, vals, (((0,), (0,)), ((), ())),
                                  preferred_element_type=F32)

        @pl.when(i == 0)
        def _():
            acc_ref[...] = contrib

        @pl.when(i > 0)
        def _():
            acc_ref[...] = acc_ref[...] + contrib

        @pl.when(i == GRID - 1)
        def _():
            stot = acc_ref[...][:, :64]
            cnt = acc_ref[...][:, 64:65]
            pooled = stot / jnp.maximum(cnt, 1.0)
            yb = jnp.dot(pooled, w3_ref[...], preferred_element_type=F32)
            yb = yb + jnp.where(cnt > 0.0, b3_ref[...], 0.0)
            out_ref[...] = (
                jnp.dot(yb, wl_ref[...], preferred_element_type=F32)
                + bl_ref[...])

    acc, out = pl.pallas_call(
        body,
        grid=(GRID,),
        in_specs=[
            pl.BlockSpec((2, RB, 128), lambda i: (0, i, 0)),
            pl.BlockSpec((RB, 128), lambda i: (i, 0)),
            pl.BlockSpec((RB, 1), lambda i: (i, 0)),
            pl.BlockSpec((RB, 1), lambda i: (i, 0)),
            pl.BlockSpec((64, 64), lambda i: (0, 0)),
            pl.BlockSpec((1, 64), lambda i: (0, 0)),
            pl.BlockSpec((64, 3), lambda i: (0, 0)),
            pl.BlockSpec((1, 3), lambda i: (0, 0)),
        ],
        out_specs=[
            pl.BlockSpec((64, 128), lambda i: (0, 0)),
            pl.BlockSpec((64, 3), lambda i: (0, 0)),
        ],
        out_shape=[
            jax.ShapeDtypeStruct((64, 128), F32),
            jax.ShapeDtypeStruct((64, 3), F32),
        ],
    )(y3, hsc2, dis, batchp, w3, b3r, wl, blr)
    del acc
    return out


def kernel(x, edge_index, batch, W1, b1, W2, b2, W3, b3, Wl, bl):
    src = edge_index[0]
    dst = edge_index[1]

    degp = _sc_deg(dst)
    degr = degp.reshape(2, NP, 1)

    xp = jnp.pad(x, ((0, NP - NN), (0, 128 - x.shape[1])))
    batchp = jnp.pad(batch, (0, NP - NN),
                     constant_values=64).reshape(NP, 1)

    dis, xs = _tc_prep(degr, xp)

    y0 = _sc_agg(xs, src, dst).reshape(2, NP, 128)
    w1p = jnp.pad(W1, ((0, 16 - W1.shape[0]), (0, 0)))
    hs1 = _tc_layer1(y0, xs, dis, w1p, b1.reshape(1, 64))

    y1 = _sc_agg(hs1, src, dst).reshape(2, NP, 128)
    hsc2 = _tc_layer2(y1, hs1, dis, W2, b2.reshape(1, 64))

    y3 = _sc_agg(hsc2, src, dst).reshape(2, NP, 128)
    out = _tc_pool(y3, hsc2, dis, batchp, W3, b3.reshape(1, 64),
                   Wl, bl.reshape(1, 3))
    return out


# speedup vs baseline: 5.9038x; 1.0120x over previous
import functools

import jax
import jax.numpy as jnp
from jax import lax
from jax.experimental import pallas as pl
from jax.experimental.pallas import tpu as pltpu
from jax.experimental.pallas import tpu_sc as plsc

NN = 100000
NP = 100352
SR = NP // 16
EE = 1600000
CH = 128
ECH = 1024
NRANGE = 8
RNG = NP // NRANGE
ACCR = RNG + 128
ZSTR = ACCR // 16
DSTR = RNG // 16
RB = 3136
GRID = NP // RB
F32 = jnp.float32

_MESH = plsc.VectorSubcoreMesh(core_axis_name="c", subcore_axis_name="s")


def _iota16():
    return lax.iota(jnp.int32, 16)


def _sc_deg(dst):
    kern = pl.kernel(
        _sc_deg_body,
        out_type=jax.ShapeDtypeStruct((2 * NP,), F32),
        mesh=_MESH,
        scratch_types=[
            pltpu.VMEM((CH,), jnp.int32),
            pltpu.VMEM((CH,), F32),
            pltpu.VMEM((SR,), F32),
            pltpu.VMEM_SHARED((NP,), F32),
        ],
    )
    return kern(dst)


def _sc_deg_body(dst_hbm, out_hbm, didx, ones_b, zb, acc):
    c = lax.axis_index("c")
    s = lax.axis_index("s")
    for j in range(CH // 16):
        ones_b[pl.ds(j * 16, 16)] = jnp.full((16,), 1.0, F32)

    def zrow(i, _):
        zb[pl.ds(i * 16, 16)] = jnp.zeros((16,), F32)
        return _

    lax.fori_loop(0, SR // 16, zrow, None)
    pltpu.sync_copy(zb, acc.at[pl.ds(s * SR, SR)])
    plsc.subcore_barrier()

    per_core = EE // 2
    per_sub = per_core // 16
    nfull = per_sub // CH
    tail = per_sub - nfull * CH
    base = c * per_core + s * per_sub

    def chunk(i, _):
        pltpu.sync_copy(dst_hbm.at[pl.ds(base + i * CH, CH)], didx)
        pltpu.sync_copy(ones_b, acc.at[didx], add=True)
        return _

    lax.fori_loop(0, nfull, chunk, None)
    pltpu.sync_copy(dst_hbm.at[pl.ds(base + nfull * CH, tail)],
                    didx.at[pl.ds(0, tail)])
    for j in range(tail // 16, CH // 16):
        didx[pl.ds(j * 16, 16)] = NN + _iota16()
    pltpu.sync_copy(ones_b, acc.at[didx], add=True)

    plsc.subcore_barrier()
    pltpu.sync_copy(acc.at[pl.ds(s * SR, SR)], zb)
    pltpu.sync_copy(zb, out_hbm.at[pl.ds(c * NP + s * SR, SR)])


BIGDST = jnp.int32(1 << 20)


def _sc_agg(table, src, dst):
    kern = pl.kernel(
        _sc_agg_body,
        out_type=jax.ShapeDtypeStruct((2, NRANGE, RNG, 128), F32),
        mesh=_MESH,
        compiler_params=pltpu.CompilerParams(needs_layout_passes=False),
        scratch_types=[
            pltpu.VMEM((ECH,), jnp.int32),
            pltpu.VMEM((ECH,), jnp.int32),
            pltpu.VMEM((2, 128), jnp.int32),
            pltpu.VMEM((2, 128), jnp.int32),
            pltpu.VMEM((128, 128), F32),
            pltpu.VMEM_SHARED((ACCR, 128), F32),
            pltpu.SemaphoreType.DMA,
        ],
    )
    return kern(table, src, dst)


def _sc_agg_body(table_hbm, src_hbm, dst_hbm, out_hbm,
                 sbuf, dbuf, cs, cd, rows, acc, sem):
    c = lax.axis_index("c")
    s = lax.axis_index("s")
    i32 = jnp.int32

    def zrow(i, _):
        for j in range(8):
            rows[i, pl.ds(j * 16, 16)] = jnp.zeros((16,), F32)
        return _

    per_core = EE // 2
    per_sub = per_core // 16
    nfull = per_sub // ECH
    tail = per_sub - nfull * ECH
    ebase = c * per_core + s * per_sub

    def group(lo, hi, sb_g, db_g, carry):
        cnt, fired = carry
        d16 = db_g
        m = (d16 >= lo) & (d16 < hi)
        pref = plsc.cumsum(jnp.where(m, 1, 0).astype(i32))
        pos = (pref + (jnp.zeros((16,), i32) + cnt) - 1) & 255
        row = (pos >> 7) & 1
        col = pos & 127
        plsc.store_scatter(cs, [row, col], sb_g, mask=m)
        plsc.store_scatter(cd, [row, col], d16 - lo, mask=m)
        cnt = cnt + pref[15]

        @pl.when(cnt - fired >= 128)
        def _():
            half = (fired // 128) & 1
            pltpu.async_copy(table_hbm.at[cs.at[half]], rows, sem).wait()
            pltpu.sync_copy(rows, acc.at[cd.at[half]], add=True)

        fired = jnp.where(cnt - fired >= 128, fired + 128, fired)
        return (cnt, fired)

    def range_body(r, _):
        lo = r * RNG
        hi = lo + RNG

        lax.fori_loop(0, 128, zrow, None)
        zb = s * ZSTR

        def zc(k, __):
            pltpu.sync_copy(rows, acc.at[pl.ds(zb + k * 128, 128)])
            return __

        lax.fori_loop(0, ZSTR // 128, zc, None)
        pltpu.sync_copy(rows.at[pl.ds(0, ZSTR % 128)],
                        acc.at[pl.ds(zb + (ZSTR // 128) * 128, ZSTR % 128)])
        plsc.subcore_barrier()

        def chunk(i, carry):
            eb = ebase + i * ECH
            pltpu.sync_copy(src_hbm.at[pl.ds(eb, ECH)], sbuf)
            pltpu.sync_copy(dst_hbm.at[pl.ds(eb, ECH)], dbuf)
            for g in range(ECH // 16):
                carry = group(lo, hi, sbuf[pl.ds(g * 16, 16)],
                              dbuf[pl.ds(g * 16, 16)], carry)
            return carry

        carry = lax.fori_loop(0, nfull, chunk, (i32(0), i32(0)))

        tb = ebase + nfull * ECH
        pltpu.sync_copy(src_hbm.at[pl.ds(tb, tail)], sbuf.at[pl.ds(0, tail)])
        pltpu.sync_copy(dst_hbm.at[pl.ds(tb, tail)], dbuf.at[pl.ds(0, tail)])
        for g in range(tail // 16, ECH // 16):
            sbuf[pl.ds(g * 16, 16)] = jnp.zeros((16,), jnp.int32)
            dbuf[pl.ds(g * 16, 16)] = jnp.zeros((16,), jnp.int32) + BIGDST

        def tailchunk(i, carry):
            for g in range(ECH // 16):
                carry = group(lo, hi, sbuf[pl.ds(g * 16, 16)],
                              dbuf[pl.ds(g * 16, 16)], carry)
            return carry

        cnt, fired = lax.fori_loop(0, 1, tailchunk, carry)

        @pl.when(cnt - fired > 0)
        def _():
            limit = jnp.zeros((16,), i32) + (fired + 128)
            for j in range(8):
                posj = (jnp.zeros((16,), i32) + cnt) + j * 16 + _iota16()
                mj = posj < limit
                pw = posj & 255
                plsc.store_scatter(cs, [(pw >> 7) & 1, pw & 127],
                                   _iota16() + (j % 2) * 16, mask=mj)
                plsc.store_scatter(cd, [(pw >> 7) & 1, pw & 127],
                                   RNG + _iota16() + (j % 2) * 16, mask=mj)
            half = (fired // 128) & 1
            pltpu.async_copy(table_hbm.at[cs.at[half]], rows, sem).wait()
            pltpu.sync_copy(rows, acc.at[cd.at[half]], add=True)

        plsc.subcore_barrier()

        db = s * DSTR

        def dc(k, __):
            pltpu.sync_copy(acc.at[pl.ds(db + k * 128, 128)], rows)
            pltpu.sync_copy(rows, out_hbm.at[c, r, pl.ds(db + k * 128, 128)])
            return __

        lax.fori_loop(0, DSTR // 128, dc, None)
        pltpu.sync_copy(acc.at[pl.ds(db + (DSTR // 128) * 128, DSTR % 128)],
                        rows.at[pl.ds(0, DSTR % 128)])
        pltpu.sync_copy(rows.at[pl.ds(0, DSTR % 128)],
                        out_hbm.at[c, r, pl.ds(db + (DSTR // 128) * 128,
                                               DSTR % 128)])
        plsc.subcore_barrier()
        return _

    lax.fori_loop(0, NRANGE, range_body, None)


def _tc_prep(degp, xp):
    def body(d_ref, x_ref, dis_ref, xs_ref):
        deg = d_ref[0] + d_ref[1] + 1.0
        dis = lax.rsqrt(deg)
        dis_ref[...] = dis
        xs_ref[...] = dis * x_ref[...]

    return pl.pallas_call(
        body,
        grid=(GRID,),
        in_specs=[
            pl.BlockSpec((2, RB, 1), lambda i: (0, i, 0)),
            pl.BlockSpec((RB, 128), lambda i: (i, 0)),
        ],
        out_specs=[
            pl.BlockSpec((RB, 1), lambda i: (i, 0)),
            pl.BlockSpec((RB, 128), lambda i: (i, 0)),
        ],
        out_shape=[
            jax.ShapeDtypeStruct((NP, 1), F32),
            jax.ShapeDtypeStruct((NP, 128), F32),
        ],
    )(degp, xp)


def _tc_layer1(y0, xs, dis, w1p, b1r):
    def body(y_ref, xs_ref, dis_ref, w_ref, b_ref, out_ref):
        dis = dis_ref[...]
        ysum = y_ref[0] + y_ref[1]
        agg = dis * (ysum[:, :16] + xs_ref[...][:, :16])
        h = jnp.maximum(
            jnp.dot(agg, w_ref[...], preferred_element_type=F32) + b_ref[...],
            0.0)
        out_ref[...] = jnp.concatenate(
            [dis * h, jnp.zeros((RB, 64), F32)], axis=1)

    return pl.pallas_call(
        body,
        grid=(GRID,),
        in_specs=[
            pl.BlockSpec((2, RB, 128), lambda i: (0, i, 0)),
            pl.BlockSpec((RB, 128), lambda i: (i, 0)),
            pl.BlockSpec((RB, 1), lambda i: (i, 0)),
            pl.BlockSpec((16, 64), lambda i: (0, 0)),
            pl.BlockSpec((1, 64), lambda i: (0, 0)),
        ],
        out_specs=pl.BlockSpec((RB, 128), lambda i: (i, 0)),
        out_shape=jax.ShapeDtypeStruct((NP, 128), F32),
    )(y0, xs, dis, w1p, b1r)


def _tc_layer2(y1, hs1, dis, w2, b2r):
    def body(y_ref, h_ref, dis_ref, w_ref, b_ref, out_ref):
        dis = dis_ref[...]
        ysum = y_ref[0] + y_ref[1]
        agg = dis * (ysum[:, :64] + h_ref[...][:, :64])
        h = jnp.maximum(
            jnp.dot(agg, w_ref[...], preferred_element_type=F32) + b_ref[...],
            0.0)
        out_ref[...] = jnp.concatenate(
            [dis * h, jnp.zeros((RB, 64), F32)], axis=1)

    return pl.pallas_call(
        body,
        grid=(GRID,),
        in_specs=[
            pl.BlockSpec((2, RB, 128), lambda i: (0, i, 0)),
            pl.BlockSpec((RB, 128), lambda i: (i, 0)),
            pl.BlockSpec((RB, 1), lambda i: (i, 0)),
            pl.BlockSpec((64, 64), lambda i: (0, 0)),
            pl.BlockSpec((1, 64), lambda i: (0, 0)),
        ],
        out_specs=pl.BlockSpec((RB, 128), lambda i: (i, 0)),
        out_shape=jax.ShapeDtypeStruct((NP, 128), F32),
    )(y1, hs1, dis, w2, b2r)


def _tc_pool(y3, hsc2, dis, batchp, w3, b3r, wl, blr):
    def body(y_ref, h_ref, dis_ref, b_ref, w3_ref, b3_ref, wl_ref, bl_ref,
             acc_ref, out_ref):
        i = pl.program_id(0)
        dis = dis_ref[...]
        ysum = y_ref[0] + y_ref[1]
        agg = dis * (ysum[:, :64] + h_ref[...][:, :64])
        vals = jnp.concatenate(
            [agg, jnp.ones((RB, 1), F32), jnp.zeros((RB, 63), F32)], axis=1)
        seg = jnp.broadcast_to(b_ref[...], (RB, 64))
        oh = (seg == lax.broadcasted_iota(jnp.int32, (RB, 64), 1)).astype(F32)
        contrib = lax.dot_general(oh, vals, (((0,), (0,)), ((), ())),
                                  preferred_element_type=F32)

        @pl.when(i == 0)
        def _():
            acc_ref[...] = contrib

        @pl.when(i > 0)
        def _():
            acc_ref[...] = acc_ref[...] + contrib

        @pl.when(i == GRID - 1)
        def _():
            stot = acc_ref[...][:, :64]
            cnt = acc_ref[...][:, 64:65]
            pooled = stot / jnp.maximum(cnt, 1.0)
            yb = jnp.dot(pooled, w3_ref[...], preferred_element_type=F32)
            yb = yb + jnp.where(cnt > 0.0, b3_ref[...], 0.0)
            out_ref[...] = (
                jnp.dot(yb, wl_ref[...], preferred_element_type=F32)
                + bl_ref[...])

    acc, out = pl.pallas_call(
        body,
        grid=(GRID,),
        in_specs=[
            pl.BlockSpec((2, RB, 128), lambda i: (0, i, 0)),
            pl.BlockSpec((RB, 128), lambda i: (i, 0)),
            pl.BlockSpec((RB, 1), lambda i: (i, 0)),
            pl.BlockSpec((RB, 1), lambda i: (i, 0)),
            pl.BlockSpec((64, 64), lambda i: (0, 0)),
            pl.BlockSpec((1, 64), lambda i: (0, 0)),
            pl.BlockSpec((64, 3), lambda i: (0, 0)),
            pl.BlockSpec((1, 3), lambda i: (0, 0)),
        ],
        out_specs=[
            pl.BlockSpec((64, 128), lambda i: (0, 0)),
            pl.BlockSpec((64, 3), lambda i: (0, 0)),
        ],
        out_shape=[
            jax.ShapeDtypeStruct((64, 128), F32),
            jax.ShapeDtypeStruct((64, 3), F32),
        ],
    )(y3, hsc2, dis, batchp, w3, b3r, wl, blr)
    del acc
    return out


def kernel(x, edge_index, batch, W1, b1, W2, b2, W3, b3, Wl, bl):
    src = edge_index[0]
    dst = edge_index[1]

    degp = _sc_deg(dst)
    degr = degp.reshape(2, NP, 1)

    xp = jnp.pad(x, ((0, NP - NN), (0, 128 - x.shape[1])))
    batchp = jnp.pad(batch, (0, NP - NN),
                     constant_values=64).reshape(NP, 1)

    dis, xs = _tc_prep(degr, xp)

    y0 = _sc_agg(xs, src, dst).reshape(2, NP, 128)
    w1p = jnp.pad(W1, ((0, 16 - W1.shape[0]), (0, 0)))
    hs1 = _tc_layer1(y0, xs, dis, w1p, b1.reshape(1, 64))

    y1 = _sc_agg(hs1, src, dst).reshape(2, NP, 128)
    hsc2 = _tc_layer2(y1, hs1, dis, W2, b2.reshape(1, 64))

    y3 = _sc_agg(hsc2, src, dst).reshape(2, NP, 128)
    out = _tc_pool(y3, hsc2, dis, batchp, W3, b3.reshape(1, 64),
                   Wl, bl.reshape(1, 3))
    return out


# speedup vs baseline: 7.2883x; 1.2345x over previous
import functools

import jax
import jax.numpy as jnp
from jax import lax
from jax.experimental import pallas as pl
from jax.experimental.pallas import tpu as pltpu
from jax.experimental.pallas import tpu_sc as plsc

NN = 100000
NP = 107136
SR = NP // 16
EE = 1600000
CH = 128
ECH = 512
NRANGE = 9
RNG = NP // NRANGE
ACCR = RNG + 128
ZSTR = ACCR // 16
DSTR = RNG // 16
RB = 4464
GRID = NP // RB
F32 = jnp.float32

_MESH = plsc.VectorSubcoreMesh(core_axis_name="c", subcore_axis_name="s")


def _iota16():
    return lax.iota(jnp.int32, 16)


def _sc_deg(dst):
    kern = pl.kernel(
        _sc_deg_body,
        out_type=jax.ShapeDtypeStruct((2 * NP,), F32),
        mesh=_MESH,
        scratch_types=[
            pltpu.VMEM((CH,), jnp.int32),
            pltpu.VMEM((CH,), F32),
            pltpu.VMEM((SR,), F32),
            pltpu.VMEM_SHARED((NP,), F32),
        ],
    )
    return kern(dst)


def _sc_deg_body(dst_hbm, out_hbm, didx, ones_b, zb, acc):
    c = lax.axis_index("c")
    s = lax.axis_index("s")
    for j in range(CH // 16):
        ones_b[pl.ds(j * 16, 16)] = jnp.full((16,), 1.0, F32)

    def zrow(i, _):
        zb[pl.ds(i * 16, 16)] = jnp.zeros((16,), F32)
        return _

    lax.fori_loop(0, SR // 16, zrow, None)
    pltpu.sync_copy(zb, acc.at[pl.ds(s * SR, SR)])
    plsc.subcore_barrier()

    per_core = EE // 2
    per_sub = per_core // 16
    nfull = per_sub // CH
    tail = per_sub - nfull * CH
    base = c * per_core + s * per_sub

    def chunk(i, _):
        pltpu.sync_copy(dst_hbm.at[pl.ds(base + i * CH, CH)], didx)
        pltpu.sync_copy(ones_b, acc.at[didx], add=True)
        return _

    lax.fori_loop(0, nfull, chunk, None)
    pltpu.sync_copy(dst_hbm.at[pl.ds(base + nfull * CH, tail)],
                    didx.at[pl.ds(0, tail)])
    for j in range(tail // 16, CH // 16):
        didx[pl.ds(j * 16, 16)] = NN + _iota16()
    pltpu.sync_copy(ones_b, acc.at[didx], add=True)

    plsc.subcore_barrier()
    pltpu.sync_copy(acc.at[pl.ds(s * SR, SR)], zb)
    pltpu.sync_copy(zb, out_hbm.at[pl.ds(c * NP + s * SR, SR)])


BIGDST = jnp.int32(1 << 20)


def _sc_agg(table, src, dst):
    kern = pl.kernel(
        _sc_agg_body,
        out_type=jax.ShapeDtypeStruct((2, NRANGE, RNG, 128), F32),
        mesh=_MESH,
        compiler_params=pltpu.CompilerParams(needs_layout_passes=False),
        scratch_types=[
            pltpu.VMEM((ECH,), jnp.int32),
            pltpu.VMEM((ECH,), jnp.int32),
            pltpu.VMEM((2, 128), jnp.int32),
            pltpu.VMEM((2, 128), jnp.int32),
            pltpu.VMEM((128, 128), F32),
            pltpu.VMEM((128, 128), F32),
            pltpu.VMEM_SHARED((ACCR, 128), F32),
            pltpu.SemaphoreType.DMA,
        ],
    )
    return kern(table, src, dst)


def _sc_agg_body(table_hbm, src_hbm, dst_hbm, out_hbm,
                 sbuf, dbuf, cs, cd, rows, rows2, acc, sem):
    c = lax.axis_index("c")
    s = lax.axis_index("s")
    i32 = jnp.int32

    def zrow(i, _):
        for j in range(8):
            rows[i, pl.ds(j * 16, 16)] = jnp.zeros((16,), F32)
        return _

    per_core = EE // 2
    per_sub = per_core // 16
    nfull = per_sub // ECH
    tail = per_sub - nfull * ECH
    ebase = c * per_core + s * per_sub

    def group(lo, hi, sb_g, db_g, carry):
        cnt, fired = carry
        d16 = db_g
        m = (d16 >= lo) & (d16 < hi)
        pref = plsc.cumsum(jnp.where(m, 1, 0).astype(i32))
        pos = (pref + (jnp.zeros((16,), i32) + cnt) - 1) & 255
        row = (pos >> 7) & 1
        col = pos & 127
        plsc.store_scatter(cs, [row, col], sb_g, mask=m)
        plsc.store_scatter(cd, [row, col], d16 - lo, mask=m)
        cnt = cnt + pref[15]

        @pl.when(cnt - fired >= 128)
        def _():
            half = (fired // 128) & 1

            @pl.when((fired >= 128) & (half == 1))
            def _():
                pltpu.make_async_copy(table_hbm.at[cs.at[0]], rows, sem).wait()
                pltpu.sync_copy(rows, acc.at[cd.at[0]], add=True)

            @pl.when((fired >= 128) & (half == 0))
            def _():
                pltpu.make_async_copy(table_hbm.at[cs.at[1]], rows2,
                                      sem).wait()
                pltpu.sync_copy(rows2, acc.at[cd.at[1]], add=True)

            @pl.when(half == 0)
            def _():
                pltpu.async_copy(table_hbm.at[cs.at[0]], rows, sem)

            @pl.when(half == 1)
            def _():
                pltpu.async_copy(table_hbm.at[cs.at[1]], rows2, sem)

        fired = jnp.where(cnt - fired >= 128, fired + 128, fired)
        return (cnt, fired)

    def range_body(r, _):
        lo = r * RNG
        hi = lo + RNG

        lax.fori_loop(0, 128, zrow, None)
        zb = s * ZSTR

        def zc(k, __):
            pltpu.sync_copy(rows, acc.at[pl.ds(zb + k * 128, 128)])
            return __

        lax.fori_loop(0, ZSTR // 128, zc, None)
        pltpu.sync_copy(rows.at[pl.ds(0, ZSTR % 128)],
                        acc.at[pl.ds(zb + (ZSTR // 128) * 128, ZSTR % 128)])
        plsc.subcore_barrier()

        def chunk(i, carry):
            eb = ebase + i * ECH
            pltpu.sync_copy(src_hbm.at[pl.ds(eb, ECH)], sbuf)
            pltpu.sync_copy(dst_hbm.at[pl.ds(eb, ECH)], dbuf)
            for g in range(ECH // 16):
                carry = group(lo, hi, sbuf[pl.ds(g * 16, 16)],
                              dbuf[pl.ds(g * 16, 16)], carry)
            return carry

        carry = lax.fori_loop(0, nfull, chunk, (i32(0), i32(0)))

        tb = ebase + nfull * ECH
        pltpu.sync_copy(src_hbm.at[pl.ds(tb, tail)], sbuf.at[pl.ds(0, tail)])
        pltpu.sync_copy(dst_hbm.at[pl.ds(tb, tail)], dbuf.at[pl.ds(0, tail)])
        for g in range(tail // 16, ECH // 16):
            sbuf[pl.ds(g * 16, 16)] = jnp.zeros((16,), jnp.int32)
            dbuf[pl.ds(g * 16, 16)] = jnp.zeros((16,), jnp.int32) + BIGDST

        def tailchunk(i, carry):
            for g in range(ECH // 16):
                carry = group(lo, hi, sbuf[pl.ds(g * 16, 16)],
                              dbuf[pl.ds(g * 16, 16)], carry)
            return carry

        cnt, fired = lax.fori_loop(0, 1, tailchunk, carry)

        @pl.when((fired >= 128) & (((fired // 128) & 1) == 1))
        def _():
            pltpu.make_async_copy(table_hbm.at[cs.at[0]], rows, sem).wait()
            pltpu.sync_copy(rows, acc.at[cd.at[0]], add=True)

        @pl.when((fired >= 128) & (((fired // 128) & 1) == 0))
        def _():
            pltpu.make_async_copy(table_hbm.at[cs.at[1]], rows2, sem).wait()
            pltpu.sync_copy(rows2, acc.at[cd.at[1]], add=True)

        @pl.when(cnt - fired > 0)
        def _():
            limit = jnp.zeros((16,), i32) + (fired + 128)
            for j in range(8):
                posj = (jnp.zeros((16,), i32) + cnt) + j * 16 + _iota16()
                mj = posj < limit
                pw = posj & 255
                plsc.store_scatter(cs, [(pw >> 7) & 1, pw & 127],
                                   _iota16() + (j % 2) * 16, mask=mj)
                plsc.store_scatter(cd, [(pw >> 7) & 1, pw & 127],
                                   RNG + _iota16() + (j % 2) * 16, mask=mj)
            half = (fired // 128) & 1
            pltpu.async_copy(table_hbm.at[cs.at[half]], rows, sem).wait()
            pltpu.sync_copy(rows, acc.at[cd.at[half]], add=True)

        plsc.subcore_barrier()

        db = s * DSTR

        def dc(k, __):
            pltpu.sync_copy(acc.at[pl.ds(db + k * 128, 128)], rows)
            pltpu.sync_copy(rows, out_hbm.at[c, r, pl.ds(db + k * 128, 128)])
            return __

        lax.fori_loop(0, DSTR // 128, dc, None)
        pltpu.sync_copy(acc.at[pl.ds(db + (DSTR // 128) * 128, DSTR % 128)],
                        rows.at[pl.ds(0, DSTR % 128)])
        pltpu.sync_copy(rows.at[pl.ds(0, DSTR % 128)],
                        out_hbm.at[c, r, pl.ds(db + (DSTR // 128) * 128,
                                               DSTR % 128)])
        plsc.subcore_barrier()
        return _

    lax.fori_loop(0, NRANGE, range_body, None)


def _tc_prep(degp, xp):
    def body(d_ref, x_ref, dis_ref, xs_ref):
        deg = d_ref[0] + d_ref[1] + 1.0
        dis = lax.rsqrt(deg)
        dis_ref[...] = dis
        xs_ref[...] = dis * x_ref[...]

    return pl.pallas_call(
        body,
        grid=(GRID,),
        in_specs=[
            pl.BlockSpec((2, RB, 1), lambda i: (0, i, 0)),
            pl.BlockSpec((RB, 128), lambda i: (i, 0)),
        ],
        out_specs=[
            pl.BlockSpec((RB, 1), lambda i: (i, 0)),
            pl.BlockSpec((RB, 128), lambda i: (i, 0)),
        ],
        out_shape=[
            jax.ShapeDtypeStruct((NP, 1), F32),
            jax.ShapeDtypeStruct((NP, 128), F32),
        ],
    )(degp, xp)


def _tc_layer1(y0, xs, dis, w1p, b1r):
    def body(y_ref, xs_ref, dis_ref, w_ref, b_ref, out_ref):
        dis = dis_ref[...]
        ysum = y_ref[0] + y_ref[1]
        agg = dis * (ysum[:, :16] + xs_ref[...][:, :16])
        h = jnp.maximum(
            jnp.dot(agg, w_ref[...], preferred_element_type=F32) + b_ref[...],
            0.0)
        out_ref[...] = jnp.concatenate(
            [dis * h, jnp.zeros((RB, 64), F32)], axis=1)

    return pl.pallas_call(
        body,
        grid=(GRID,),
        in_specs=[
            pl.BlockSpec((2, RB, 128), lambda i: (0, i, 0)),
            pl.BlockSpec((RB, 128), lambda i: (i, 0)),
            pl.BlockSpec((RB, 1), lambda i: (i, 0)),
            pl.BlockSpec((16, 64), lambda i: (0, 0)),
            pl.BlockSpec((1, 64), lambda i: (0, 0)),
        ],
        out_specs=pl.BlockSpec((RB, 128), lambda i: (i, 0)),
        out_shape=jax.ShapeDtypeStruct((NP, 128), F32),
    )(y0, xs, dis, w1p, b1r)


def _tc_layer2(y1, hs1, dis, w2, b2r):
    def body(y_ref, h_ref, dis_ref, w_ref, b_ref, out_ref):
        dis = dis_ref[...]
        ysum = y_ref[0] + y_ref[1]
        agg = dis * (ysum[:, :64] + h_ref[...][:, :64])
        h = jnp.maximum(
            jnp.dot(agg, w_ref[...], preferred_element_type=F32) + b_ref[...],
            0.0)
        out_ref[...] = jnp.concatenate(
            [dis * h, jnp.zeros((RB, 64), F32)], axis=1)

    return pl.pallas_call(
        body,
        grid=(GRID,),
        in_specs=[
            pl.BlockSpec((2, RB, 128), lambda i: (0, i, 0)),
            pl.BlockSpec((RB, 128), lambda i: (i, 0)),
            pl.BlockSpec((RB, 1), lambda i: (i, 0)),
            pl.BlockSpec((64, 64), lambda i: (0, 0)),
            pl.BlockSpec((1, 64), lambda i: (0, 0)),
        ],
        out_specs=pl.BlockSpec((RB, 128), lambda i: (i, 0)),
        out_shape=jax.ShapeDtypeStruct((NP, 128), F32),
    )(y1, hs1, dis, w2, b2r)


def _tc_pool(y3, hsc2, dis, batchp, w3, b3r, wl, blr):
    def body(y_ref, h_ref, dis_ref, b_ref, w3_ref, b3_ref, wl_ref, bl_ref,
             acc_ref, out_ref):
        i = pl.program_id(0)
        dis = dis_ref[...]
        ysum = y_ref[0] + y_ref[1]
        agg = dis * (ysum[:, :64] + h_ref[...][:, :64])
        vals = jnp.concatenate(
            [agg, jnp.ones((RB, 1), F32), jnp.zeros((RB, 63), F32)], axis=1)
        seg = jnp.broadcast_to(b_ref[...], (RB, 64))
        oh = (seg == lax.broadcasted_iota(jnp.int32, (RB, 64), 1)).astype(F32)
        contrib = lax.dot_general(oh, vals, (((0,), (0,)), ((), ())),
                                  preferred_element_type=F32)

        @pl.when(i == 0)
        def _():
            acc_ref[...] = contrib

        @pl.when(i > 0)
        def _():
            acc_ref[...] = acc_ref[...] + contrib

        @pl.when(i == GRID - 1)
        def _():
            stot = acc_ref[...][:, :64]
            cnt = acc_ref[...][:, 64:65]
            pooled = stot / jnp.maximum(cnt, 1.0)
            yb = jnp.dot(pooled, w3_ref[...], preferred_element_type=F32)
            yb = yb + jnp.where(cnt > 0.0, b3_ref[...], 0.0)
            out_ref[...] = (
                jnp.dot(yb, wl_ref[...], preferred_element_type=F32)
                + bl_ref[...])

    acc, out = pl.pallas_call(
        body,
        grid=(GRID,),
        in_specs=[
            pl.BlockSpec((2, RB, 128), lambda i: (0, i, 0)),
            pl.BlockSpec((RB, 128), lambda i: (i, 0)),
            pl.BlockSpec((RB, 1), lambda i: (i, 0)),
            pl.BlockSpec((RB, 1), lambda i: (i, 0)),
            pl.BlockSpec((64, 64), lambda i: (0, 0)),
            pl.BlockSpec((1, 64), lambda i: (0, 0)),
            pl.BlockSpec((64, 3), lambda i: (0, 0)),
            pl.BlockSpec((1, 3), lambda i: (0, 0)),
        ],
        out_specs=[
            pl.BlockSpec((64, 128), lambda i: (0, 0)),
            pl.BlockSpec((64, 3), lambda i: (0, 0)),
        ],
        out_shape=[
            jax.ShapeDtypeStruct((64, 128), F32),
            jax.ShapeDtypeStruct((64, 3), F32),
        ],
    )(y3, hsc2, dis, batchp, w3, b3r, wl, blr)
    del acc
    return out


def kernel(x, edge_index, batch, W1, b1, W2, b2, W3, b3, Wl, bl):
    src = edge_index[0]
    dst = edge_index[1]

    degp = _sc_deg(dst)
    degr = degp.reshape(2, NP, 1)

    xp = jnp.pad(x, ((0, NP - NN), (0, 128 - x.shape[1])))
    batchp = jnp.pad(batch, (0, NP - NN),
                     constant_values=64).reshape(NP, 1)

    dis, xs = _tc_prep(degr, xp)

    y0 = _sc_agg(xs, src, dst).reshape(2, NP, 128)
    w1p = jnp.pad(W1, ((0, 16 - W1.shape[0]), (0, 0)))
    hs1 = _tc_layer1(y0, xs, dis, w1p, b1.reshape(1, 64))

    y1 = _sc_agg(hs1, src, dst).reshape(2, NP, 128)
    hsc2 = _tc_layer2(y1, hs1, dis, W2, b2.reshape(1, 64))

    y3 = _sc_agg(hsc2, src, dst).reshape(2, NP, 128)
    out = _tc_pool(y3, hsc2, dis, batchp, W3, b3.reshape(1, 64),
                   Wl, bl.reshape(1, 3))
    return out
